# baseline (speedup 1.0000x reference)
import functools

import jax
import jax.numpy as jnp
from jax import lax
from jax.experimental import pallas as pl
from jax.experimental.pallas import tpu as pltpu
from jax.experimental.pallas import tpu_sc as plsc

_K_RATIO = 0.1

_NBINS = 256
_STRIDE = _NBINS + 1
_HISTW = 16 * _STRIDE
_PASSES = ((23, 8), (15, 8), (7, 8), (0, 7))


def _row_thresh_bits(row_ref, hist_ref, k, n):
    lane = lax.iota(jnp.int32, 16)
    lane_off = lane * _STRIDE
    ones = jnp.ones((16,), jnp.int32)
    zeros = jnp.zeros((16,), jnp.int32)
    kk = jnp.int32(k)
    prefix = jnp.int32(0)

    for sh, nb in _PASSES:
        def clr(i, _):
            hist_ref[pl.ds(i * 16, 16)] = zeros
            return 0

        lax.fori_loop(0, _HISTW // 16, clr, 0)

        def scat(i, _, sh=sh, nb=nb, prefix=prefix):
            for u in range(8):
                v = row_ref[pl.ds(i * 128 + u * 16, 16)]
                b = v & jnp.int32(0x7FFFFFFF)
                hi = b >> sh
                bucket = hi & jnp.int32((1 << nb) - 1)
                m = (hi >> nb) == prefix
                plsc.addupdate_scatter(hist_ref, [lane_off + bucket], ones, mask=m)
            return 0

        lax.fori_loop(0, n // 128, scat, 0)

        def scan_chunk(i, st, kk=kk):
            carry, bcount, gtb = st
            c = jnp.int32(_NBINS // 16 - 1) - i
            tot = zeros
            for l in range(16):
                tot = tot + hist_ref[pl.ds(l * _STRIDE + c * 16, 16)]
            sfx = jnp.cumsum(jnp.flip(tot, 0)) + carry
            bcount = bcount + jnp.sum((sfx >= kk).astype(jnp.int32))
            gtb = jnp.maximum(gtb, jnp.max(jnp.where(sfx < kk, sfx, 0)))
            return sfx[15], bcount, gtb

        _, bcount, gtb = lax.fori_loop(
            0, _NBINS // 16, scan_chunk,
            (jnp.int32(0), jnp.int32(0), jnp.int32(0)),
        )
        b_sel = bcount - 1
        kk = kk - gtb
        prefix = (prefix << nb) | b_sel
    return prefix


def _make_sc_thresh(m, n, k):
    mesh = plsc.VectorSubcoreMesh(core_axis_name="c", subcore_axis_name="s")
    rows_per = m // 32

    @functools.partial(
        pl.kernel,
        mesh=mesh,
        out_type=jax.ShapeDtypeStruct((32, 16), jnp.int32),
        compiler_params=pltpu.CompilerParams(needs_layout_passes=False),
        scratch_types=[
            pltpu.VMEM((n,), jnp.int32),
            pltpu.VMEM((n,), jnp.int32),
            pltpu.VMEM((_HISTW,), jnp.int32),
            pltpu.VMEM((16,), jnp.int32),
            pltpu.SemaphoreType.DMA,
            pltpu.SemaphoreType.DMA,
        ],
    )
    def sc_thresh(x_hbm, out_hbm, row_a, row_b, hist, tvec, sem_a, sem_b):
        wid = lax.axis_index("c") * 16 + lax.axis_index("s")
        base = wid * rows_per
        bufs = (row_a, row_b)
        sems = (sem_a, sem_b)
        lane = lax.iota(jnp.int32, 16)
        tv = jnp.zeros((16,), jnp.int32)
        h = pltpu.async_copy(x_hbm.at[base], bufs[0], sems[0])
        for j in range(rows_per):
            if j + 1 < rows_per:
                h_next = pltpu.async_copy(
                    x_hbm.at[base + j + 1], bufs[(j + 1) % 2], sems[(j + 1) % 2]
                )
            h.wait()
            t = _row_thresh_bits(bufs[j % 2], hist, k, n)
            tv = jnp.where(lane == j, t, tv)
            if j + 1 < rows_per:
                h = h_next
        tvec[...] = tv
        pltpu.sync_copy(tvec, out_hbm.at[wid])

    return sc_thresh


def _masknorm_body(x_ref, t_ref, o_ref):
    x = x_ref[...]
    bits = lax.bitcast_convert_type(x, jnp.int32) & jnp.int32(0x7FFFFFFF)
    xm = jnp.where(bits >= t_ref[...], x, 0.0)
    ss = jnp.sum(xm * xm, axis=-1, keepdims=True)
    o_ref[...] = xm / (jnp.sqrt(ss) + 1e-6)


@jax.jit
def kernel(x):
    m, n = x.shape
    k = int(_K_RATIO * n)
    rows_per = m // 32
    xi = lax.bitcast_convert_type(x, jnp.int32)
    tb = _make_sc_thresh(m, n, k)(xi)
    tbits = tb[:, :rows_per].reshape(m, 1)
    r = 8
    return pl.pallas_call(
        _masknorm_body,
        grid=(m // r,),
        in_specs=[
            pl.BlockSpec((r, n), lambda i: (i, 0)),
            pl.BlockSpec((r, 1), lambda i: (i, 0)),
        ],
        out_specs=pl.BlockSpec((r, n), lambda i: (i, 0)),
        out_shape=jax.ShapeDtypeStruct((m, n), jnp.float32),
    )(x, tbits)


# speedup vs baseline: 1.0889x; 1.0889x over previous
import functools

import jax
import jax.numpy as jnp
from jax import lax
from jax.experimental import pallas as pl
from jax.experimental.pallas import tpu as pltpu
from jax.experimental.pallas import tpu_sc as plsc

_K_RATIO = 0.1

_NBINS = 256
_HISTW = 16 * _NBINS
_PASSES = ((23, 8), (15, 8), (7, 8), (0, 7))


def _row_thresh_bits(row_ref, hist_ref, k, n):
    lane = lax.iota(jnp.int32, 16)
    ones = jnp.ones((16,), jnp.int32)
    zeros = jnp.zeros((16,), jnp.int32)
    kk = jnp.int32(k)
    prefix = jnp.int32(0)

    for sh, nb in _PASSES:
        def scat(i, _, sh=sh, nb=nb, prefix=prefix):
            for u in range(16):
                v = row_ref[pl.ds(i * 256 + u * 16, 16)]
                b = plsc.bitcast(v, jnp.int32) & jnp.int32(0x7FFFFFFF)
                hi = b >> sh
                bucket = hi & jnp.int32((1 << nb) - 1)
                m = (hi >> nb) == prefix
                plsc.addupdate_scatter(
                    hist_ref, [(bucket << 4) | lane], ones, mask=m
                )
            return 0

        lax.fori_loop(0, n // 256, scat, 0)

        def scan_b(i, st, kk=kk):
            carry, bcount, gtb = st
            for u in range(4):
                c = jnp.int32(_NBINS - 1) - (i * 4 + u)
                v = hist_ref[pl.ds(c * 16, 16)]
                hist_ref[pl.ds(c * 16, 16)] = zeros
                carry = carry + jnp.sum(v)
                bcount = bcount + (carry >= kk).astype(jnp.int32)
                gtb = jnp.maximum(gtb, jnp.where(carry < kk, carry, 0))
            return carry, bcount, gtb

        _, bcount, gtb = lax.fori_loop(
            0, _NBINS // 4, scan_b,
            (jnp.int32(0), jnp.int32(0), jnp.int32(0)),
        )
        kk = kk - gtb
        prefix = (prefix << nb) | (bcount - 1)
    return prefix


def _mask_scale_row(row_ref, tbits, k, n):
    signmask = jnp.int32(0x7FFFFFFF)
    fzeros = jnp.zeros((16,), jnp.float32)

    def ssq(i, accs):
        a0, a1, a2, a3 = accs
        new = []
        for u, a in enumerate((a0, a1, a2, a3)):
            for w in range(4):
                v = row_ref[pl.ds(i * 256 + (u * 4 + w) * 16, 16)]
                b = plsc.bitcast(v, jnp.int32) & signmask
                xm = jnp.where(b >= tbits, v, 0.0)
                a = a + xm * xm
            new.append(a)
        return tuple(new)

    a0, a1, a2, a3 = lax.fori_loop(
        0, n // 256, ssq, (fzeros, fzeros, fzeros, fzeros)
    )
    ss = jnp.sum(a0 + a1 + a2 + a3)

    ssv = jnp.full((16,), ss, jnp.float32)
    y = plsc.bitcast(
        (plsc.bitcast(ssv, jnp.int32) >> 1) + jnp.int32(0x1FBD1DF5), jnp.float32
    )
    for _ in range(3):
        y = 0.5 * (y + ssv / y)
    inv = 1.0 / (y + 1e-6)
    inv = inv[0]

    def scale(i, _):
        for u in range(16):
            sl = pl.ds(i * 256 + u * 16, 16)
            v = row_ref[sl]
            b = plsc.bitcast(v, jnp.int32) & signmask
            xm = jnp.where(b >= tbits, v, 0.0)
            row_ref[sl] = xm * inv
        return 0

    lax.fori_loop(0, n // 256, scale, 0)


def _make_sc_kernel(m, n, k):
    mesh = plsc.VectorSubcoreMesh(core_axis_name="c", subcore_axis_name="s")
    rows_per = m // 32

    @functools.partial(
        pl.kernel,
        mesh=mesh,
        out_type=jax.ShapeDtypeStruct((m, n), jnp.float32),
        compiler_params=pltpu.CompilerParams(needs_layout_passes=False),
        scratch_types=[
            pltpu.VMEM((n,), jnp.float32),
            pltpu.VMEM((n,), jnp.float32),
            pltpu.VMEM((_HISTW,), jnp.int32),
            pltpu.SemaphoreType.DMA,
            pltpu.SemaphoreType.DMA,
            pltpu.SemaphoreType.DMA,
            pltpu.SemaphoreType.DMA,
        ],
    )
    def sc_kernel(x_hbm, out_hbm, row_a, row_b, hist, si_a, si_b, so_a, so_b):
        wid = lax.axis_index("c") * 16 + lax.axis_index("s")
        base = wid * rows_per
        bufs = (row_a, row_b)
        sin = (si_a, si_b)
        sout = (so_a, so_b)
        zeros = jnp.zeros((16,), jnp.int32)

        def clr(i, _):
            for u in range(8):
                hist[pl.ds(i * 128 + u * 16, 16)] = zeros
            return 0

        lax.fori_loop(0, _HISTW // 128, clr, 0)

        h_in = [None] * rows_per
        h_out = [None] * rows_per
        h_in[0] = pltpu.async_copy(x_hbm.at[base], bufs[0], sin[0])
        if rows_per > 1:
            h_in[1] = pltpu.async_copy(x_hbm.at[base + 1], bufs[1], sin[1])
        for j in range(rows_per):
            h_in[j].wait()
            buf = bufs[j % 2]
            t = _row_thresh_bits(buf, hist, k, n)
            _mask_scale_row(buf, t, k, n)
            h_out[j] = pltpu.async_copy(buf, out_hbm.at[base + j], sout[j % 2])
            if j + 2 < rows_per:
                h_out[j].wait()
                h_in[j + 2] = pltpu.async_copy(
                    x_hbm.at[base + j + 2], bufs[j % 2], sin[j % 2]
                )
        for j in range(max(0, rows_per - 2), rows_per):
            h_out[j].wait()

    return sc_kernel


@jax.jit
def kernel(x):
    m, n = x.shape
    k = int(_K_RATIO * n)
    return _make_sc_kernel(m, n, k)(x)


# speedup vs baseline: 2.9661x; 2.7239x over previous
import functools

import jax
import jax.numpy as jnp
from jax import lax
from jax.experimental import pallas as pl
from jax.experimental.pallas import tpu as pltpu
from jax.experimental.pallas import tpu_sc as plsc

_K_RATIO = 0.1

_NBINS = 256
_HISTW = 16 * _NBINS
_PASSES = ((23, 8), (15, 8), (7, 8), (0, 7))


def _row_thresh_bits(row_ref, hist_ref, k, n):
    lane = lax.iota(jnp.int32, 16)
    ones = jnp.ones((16,), jnp.int32)
    zeros = jnp.zeros((16,), jnp.int32)
    kk = jnp.int32(k)
    prefix = jnp.int32(0)

    for sh, nb in _PASSES:
        def scat(i, _, sh=sh, nb=nb, prefix=prefix):
            base = i * 128
            vs = [row_ref[pl.ds(base + u * 16, 16)] for u in range(8)]
            idxs, ms = [], []
            for v in vs:
                b = plsc.bitcast(v, jnp.int32) & jnp.int32(0x7FFFFFFF)
                hi = b >> sh
                idxs.append((((hi & jnp.int32((1 << nb) - 1)) << 4)) | lane)
                ms.append((hi >> nb) == prefix)
            for idx, m in zip(idxs, ms):
                plsc.addupdate_scatter(hist_ref, [idx], ones, mask=m)
            return 0

        lax.fori_loop(0, n // 128, scat, 0)

        def scan_b(i, st, kk=kk):
            carry, bcount, gtb = st
            for u in range(4):
                c = jnp.int32(_NBINS - 1) - (i * 4 + u)
                v = hist_ref[pl.ds(c * 16, 16)]
                hist_ref[pl.ds(c * 16, 16)] = zeros
                carry = carry + jnp.sum(v)
                bcount = bcount + (carry >= kk).astype(jnp.int32)
                gtb = jnp.maximum(gtb, jnp.where(carry < kk, carry, 0))
            return carry, bcount, gtb

        _, bcount, gtb = lax.fori_loop(
            0, _NBINS // 4, scan_b,
            (jnp.int32(0), jnp.int32(0), jnp.int32(0)),
        )
        kk = kk - gtb
        prefix = (prefix << nb) | (bcount - 1)
    return prefix


def _mask_scale_row(row_ref, tbits, k, n):
    signmask = jnp.int32(0x7FFFFFFF)
    fzeros = jnp.zeros((16,), jnp.float32)

    def ssq(i, accs):
        base = i * 128
        vs = [row_ref[pl.ds(base + u * 16, 16)] for u in range(8)]
        new = []
        for v, a in zip(vs, accs):
            b = plsc.bitcast(v, jnp.int32) & signmask
            xm = jnp.where(b >= tbits, v, 0.0)
            new.append(a + xm * xm)
        return tuple(new)

    accs = lax.fori_loop(0, n // 128, ssq, (fzeros,) * 8)
    ss = jnp.sum(accs[0] + accs[1] + accs[2] + accs[3]
                 + accs[4] + accs[5] + accs[6] + accs[7])

    ssv = jnp.full((16,), ss, jnp.float32)
    y = plsc.bitcast(
        (plsc.bitcast(ssv, jnp.int32) >> 1) + jnp.int32(0x1FBD1DF5), jnp.float32
    )
    for _ in range(3):
        y = 0.5 * (y + ssv / y)
    inv = 1.0 / (y + 1e-6)
    inv = inv[0]

    def scale(i, _):
        base = i * 128
        vs = [row_ref[pl.ds(base + u * 16, 16)] for u in range(8)]
        outs = []
        for v in vs:
            b = plsc.bitcast(v, jnp.int32) & signmask
            outs.append(jnp.where(b >= tbits, v, 0.0) * inv)
        for u, o in enumerate(outs):
            row_ref[pl.ds(base + u * 16, 16)] = o
        return 0

    lax.fori_loop(0, n // 128, scale, 0)


def _make_sc_kernel(m, n, k):
    mesh = plsc.VectorSubcoreMesh(core_axis_name="c", subcore_axis_name="s")
    rows_per = m // 32

    @functools.partial(
        pl.kernel,
        mesh=mesh,
        out_type=jax.ShapeDtypeStruct((m, n), jnp.float32),
        compiler_params=pltpu.CompilerParams(needs_layout_passes=False),
        scratch_types=[
            pltpu.VMEM((n,), jnp.float32),
            pltpu.VMEM((n,), jnp.float32),
            pltpu.VMEM((_HISTW,), jnp.int32),
            pltpu.SemaphoreType.DMA,
            pltpu.SemaphoreType.DMA,
            pltpu.SemaphoreType.DMA,
            pltpu.SemaphoreType.DMA,
        ],
    )
    def sc_kernel(x_hbm, out_hbm, row_a, row_b, hist, si_a, si_b, so_a, so_b):
        wid = lax.axis_index("c") * 16 + lax.axis_index("s")
        base = wid * rows_per
        bufs = (row_a, row_b)
        sin = (si_a, si_b)
        sout = (so_a, so_b)
        zeros = jnp.zeros((16,), jnp.int32)

        def clr(i, _):
            for u in range(8):
                hist[pl.ds(i * 128 + u * 16, 16)] = zeros
            return 0

        lax.fori_loop(0, _HISTW // 128, clr, 0)

        h_in = [None] * rows_per
        h_out = [None] * rows_per
        h_in[0] = pltpu.async_copy(x_hbm.at[base], bufs[0], sin[0])
        if rows_per > 1:
            h_in[1] = pltpu.async_copy(x_hbm.at[base + 1], bufs[1], sin[1])
        for j in range(rows_per):
            h_in[j].wait()
            buf = bufs[j % 2]
            t = _row_thresh_bits(buf, hist, k, n)
            _mask_scale_row(buf, t, k, n)
            h_out[j] = pltpu.async_copy(buf, out_hbm.at[base + j], sout[j % 2])
            if j + 2 < rows_per:
                h_out[j].wait()
                h_in[j + 2] = pltpu.async_copy(
                    x_hbm.at[base + j + 2], bufs[j % 2], sin[j % 2]
                )
        for j in range(max(0, rows_per - 2), rows_per):
            h_out[j].wait()

    return sc_kernel


@jax.jit
def kernel(x):
    m, n = x.shape
    k = int(_K_RATIO * n)
    return _make_sc_kernel(m, n, k)(x)


# speedup vs baseline: 3.2675x; 1.1016x over previous
import functools

import jax
import jax.numpy as jnp
from jax import lax
from jax.experimental import pallas as pl
from jax.experimental.pallas import tpu as pltpu
from jax.experimental.pallas import tpu_sc as plsc

_K_RATIO = 0.1

_NBINS = 256
_HISTW = 16 * _NBINS
_SENT = 0x7FFFFFFF


def _select_pass(read_vreg, nvec128, hist_ref, kk, prefix, sh, nb, lane,
                 ones, zeros):
    def scat(i, _):
        bs = [read_vreg(i, u) for u in range(8)]
        idxs, ms = [], []
        for b in bs:
            hi = b >> sh
            idxs.append((((hi & jnp.int32((1 << nb) - 1)) << 4)) | lane)
            ms.append((hi >> nb) == prefix)
        for idx, m in zip(idxs, ms):
            plsc.addupdate_scatter(hist_ref, [idx], ones, mask=m)
        return 0

    lax.fori_loop(0, nvec128, scat, 0)

    def scan_b(i, st):
        carry, bcount, gtb = st
        for u in range(4):
            c = jnp.int32(_NBINS - 1) - (i * 4 + u)
            v = hist_ref[pl.ds(c * 16, 16)]
            hist_ref[pl.ds(c * 16, 16)] = zeros
            carry = carry + jnp.sum(v)
            bcount = bcount + (carry >= kk).astype(jnp.int32)
            gtb = jnp.maximum(gtb, jnp.where(carry < kk, carry, 0))
        return carry, bcount, gtb

    _, bcount, gtb = lax.fori_loop(
        0, _NBINS // 4, scan_b,
        (jnp.int32(0), jnp.int32(0), jnp.int32(0)),
    )
    return kk - gtb, (prefix << nb) | (bcount - 1)


def _row_thresh_bits(row_ref, hist_ref, cand_ref, k, n):
    lane = lax.iota(jnp.int32, 16)
    ones = jnp.ones((16,), jnp.int32)
    zeros = jnp.zeros((16,), jnp.int32)
    kk = jnp.int32(k)
    prefix = jnp.int32(0)

    def read_row(i, u):
        v = row_ref[pl.ds(i * 128 + u * 16, 16)]
        return plsc.bitcast(v, jnp.int32) & jnp.int32(0x7FFFFFFF)

    kk, prefix = _select_pass(
        read_row, n // 128, hist_ref, kk, prefix, 23, 8, lane, ones, zeros
    )

    def cpt(i, off):
        bs = [read_row(i, u) for u in range(8)]
        ms = [(b >> 23) == prefix for b in bs]
        for b, m in zip(bs, ms):
            plsc.store_compressed(cand_ref.at[pl.ds(off, 16)], b, mask=m)
            off = off + plsc.all_reduce_population_count(m)[0]
        return off

    c1 = lax.fori_loop(0, n // 128, cpt, jnp.int32(0))
    sent = jnp.full((16,), _SENT, jnp.int32)
    for u in range(8):
        cand_ref[pl.ds(c1 + u * 16, 16)] = sent
    nit = (c1 + jnp.int32(127)) >> 7

    def read_cand(i, u):
        return cand_ref[pl.ds(i * 128 + u * 16, 16)]

    for sh, nb in ((15, 8), (7, 8), (0, 7)):
        kk, prefix = _select_pass(
            read_cand, nit, hist_ref, kk, prefix, sh, nb, lane, ones, zeros
        )
    return prefix


def _mask_scale_row(row_ref, tbits, k, n):
    signmask = jnp.int32(0x7FFFFFFF)
    fzeros = jnp.zeros((16,), jnp.float32)

    def ssq(i, accs):
        base = i * 128
        vs = [row_ref[pl.ds(base + u * 16, 16)] for u in range(8)]
        new = []
        for v, a in zip(vs, accs):
            b = plsc.bitcast(v, jnp.int32) & signmask
            xm = jnp.where(b >= tbits, v, 0.0)
            new.append(a + xm * xm)
        return tuple(new)

    accs = lax.fori_loop(0, n // 128, ssq, (fzeros,) * 8)
    ss = jnp.sum(accs[0] + accs[1] + accs[2] + accs[3]
                 + accs[4] + accs[5] + accs[6] + accs[7])

    ssv = jnp.full((16,), ss, jnp.float32)
    y = plsc.bitcast(
        (plsc.bitcast(ssv, jnp.int32) >> 1) + jnp.int32(0x1FBD1DF5), jnp.float32
    )
    for _ in range(3):
        y = 0.5 * (y + ssv / y)
    inv = 1.0 / (y + 1e-6)
    inv = inv[0]

    def scale(i, _):
        base = i * 128
        vs = [row_ref[pl.ds(base + u * 16, 16)] for u in range(8)]
        outs = []
        for v in vs:
            b = plsc.bitcast(v, jnp.int32) & signmask
            outs.append(jnp.where(b >= tbits, v, 0.0) * inv)
        for u, o in enumerate(outs):
            row_ref[pl.ds(base + u * 16, 16)] = o
        return 0

    lax.fori_loop(0, n // 128, scale, 0)


def _make_sc_kernel(m, n, k):
    mesh = plsc.VectorSubcoreMesh(core_axis_name="c", subcore_axis_name="s")
    rows_per = m // 32

    @functools.partial(
        pl.kernel,
        mesh=mesh,
        out_type=jax.ShapeDtypeStruct((m, n), jnp.float32),
        compiler_params=pltpu.CompilerParams(needs_layout_passes=False),
        scratch_types=[
            pltpu.VMEM((n,), jnp.float32),
            pltpu.VMEM((n,), jnp.float32),
            pltpu.VMEM((_HISTW,), jnp.int32),
            pltpu.VMEM((n + 128,), jnp.int32),
            pltpu.SemaphoreType.DMA,
            pltpu.SemaphoreType.DMA,
            pltpu.SemaphoreType.DMA,
            pltpu.SemaphoreType.DMA,
        ],
    )
    def sc_kernel(x_hbm, out_hbm, row_a, row_b, hist, cand,
                  si_a, si_b, so_a, so_b):
        wid = lax.axis_index("c") * 16 + lax.axis_index("s")
        base = wid * rows_per
        bufs = (row_a, row_b)
        sin = (si_a, si_b)
        sout = (so_a, so_b)
        zeros = jnp.zeros((16,), jnp.int32)

        def clr(i, _):
            for u in range(8):
                hist[pl.ds(i * 128 + u * 16, 16)] = zeros
            return 0

        lax.fori_loop(0, _HISTW // 128, clr, 0)

        h_in = [None] * rows_per
        h_out = [None] * rows_per
        h_in[0] = pltpu.async_copy(x_hbm.at[base], bufs[0], sin[0])
        if rows_per > 1:
            h_in[1] = pltpu.async_copy(x_hbm.at[base + 1], bufs[1], sin[1])
        for j in range(rows_per):
            h_in[j].wait()
            buf = bufs[j % 2]
            t = _row_thresh_bits(buf, hist, cand, k, n)
            _mask_scale_row(buf, t, k, n)
            h_out[j] = pltpu.async_copy(buf, out_hbm.at[base + j], sout[j % 2])
            if j + 2 < rows_per:
                h_out[j].wait()
                h_in[j + 2] = pltpu.async_copy(
                    x_hbm.at[base + j + 2], bufs[j % 2], sin[j % 2]
                )
        for j in range(max(0, rows_per - 2), rows_per):
            h_out[j].wait()

    return sc_kernel


@jax.jit
def kernel(x):
    m, n = x.shape
    k = int(_K_RATIO * n)
    return _make_sc_kernel(m, n, k)(x)


# speedup vs baseline: 3.5629x; 1.0904x over previous
import functools

import jax
import jax.numpy as jnp
from jax import lax
from jax.experimental import pallas as pl
from jax.experimental.pallas import tpu as pltpu
from jax.experimental.pallas import tpu_sc as plsc

_K_RATIO = 0.1

_NBINS = 256
_HISTW = 16 * _NBINS
_SENT = 0x7FFFFFFF


def _select_pass(read_vreg, ngroups, unroll, hist_ref, kk, prefix, sh, nb,
                 lane, ones, zeros):
    def scat(i, _):
        bs = [read_vreg(i, u) for u in range(unroll)]
        idxs, ms = [], []
        for b in bs:
            hi = b >> sh
            idxs.append((((hi & jnp.int32((1 << nb) - 1)) << 4)) | lane)
            ms.append((hi >> nb) == prefix)
        for idx, m in zip(idxs, ms):
            plsc.addupdate_scatter(hist_ref, [idx], ones, mask=m)
        return 0

    lax.fori_loop(0, ngroups, scat, 0)

    def scan_b(i, st):
        carry, bcount, gtb = st
        for u in range(4):
            c = jnp.int32(_NBINS - 1) - (i * 4 + u)
            v = hist_ref[pl.ds(c * 16, 16)]
            hist_ref[pl.ds(c * 16, 16)] = zeros
            carry = carry + jnp.sum(v)
            bcount = bcount + (carry >= kk).astype(jnp.int32)
            gtb = jnp.maximum(gtb, jnp.where(carry < kk, carry, 0))
        return carry, bcount, gtb

    _, bcount, gtb = lax.fori_loop(
        0, _NBINS // 4, scan_b,
        (jnp.int32(0), jnp.int32(0), jnp.int32(0)),
    )
    return kk - gtb, (prefix << nb) | (bcount - 1)


def _row_thresh_ss(row_ref, hist_ref, cand_ref, k, n):
    lane = lax.iota(jnp.int32, 16)
    ones = jnp.ones((16,), jnp.int32)
    zeros = jnp.zeros((16,), jnp.int32)
    fzeros = jnp.zeros((16,), jnp.float32)
    kk = jnp.int32(k)
    prefix = jnp.int32(0)

    def read_row(i, u):
        v = row_ref[pl.ds(i * 256 + u * 16, 16)]
        return plsc.bitcast(v, jnp.int32) & jnp.int32(0x7FFFFFFF)

    kk, prefix = _select_pass(
        read_row, n // 256, 16, hist_ref, kk, prefix, 23, 8, lane, ones,
        zeros
    )

    def cpt(i, st):
        off = st[0]
        accs = st[1:]
        bs = [
            plsc.bitcast(row_ref[pl.ds(i * 128 + u * 16, 16)], jnp.int32)
            & jnp.int32(0x7FFFFFFF)
            for u in range(8)
        ]
        ms = [(b >> 23) == prefix for b in bs]
        pcs = [plsc.all_reduce_population_count(m)[0] for m in ms]
        offs = []
        for pc in pcs:
            offs.append(off)
            off = off + pc
        new = []
        for b, a in zip(bs, accs):
            hi = plsc.bitcast(b, jnp.float32)
            xm = jnp.where((b >> 23) > prefix, hi, 0.0)
            new.append(a + xm * xm)
        for b, m, o in zip(bs, ms, offs):
            plsc.store_compressed(cand_ref.at[pl.ds(o, 16)], b, mask=m)
        return (off, *new)

    st = lax.fori_loop(0, n // 128, cpt, (jnp.int32(0),) + (fzeros,) * 8)
    c1 = st[0]
    ss_hi = st[1] + st[2] + st[3] + st[4] + st[5] + st[6] + st[7] + st[8]
    sent = jnp.full((16,), _SENT, jnp.int32)
    for u in range(8):
        cand_ref[pl.ds(c1 + u * 16, 16)] = sent
    nit = (c1 + jnp.int32(127)) >> 7

    def read_cand(i, u):
        return cand_ref[pl.ds(i * 128 + u * 16, 16)]

    for sh, nb in ((15, 8), (7, 8), (0, 7)):
        kk, prefix = _select_pass(
            read_cand, nit, 8, hist_ref, kk, prefix, sh, nb, lane, ones,
            zeros
        )

    def cssq(i, accs):
        bs = [read_cand(i, u) for u in range(8)]
        new = []
        for b, a in zip(bs, accs):
            hi = plsc.bitcast(b, jnp.float32)
            keep = (b >= prefix) & (b < jnp.int32(_SENT))
            xm = jnp.where(keep, hi, 0.0)
            new.append(a + xm * xm)
        return tuple(new)

    accs = lax.fori_loop(0, nit, cssq, (fzeros,) * 8)
    ss_cand = (accs[0] + accs[1] + accs[2] + accs[3]
               + accs[4] + accs[5] + accs[6] + accs[7])
    return prefix, jnp.sum(ss_hi + ss_cand)


def _mask_scale_row(row_ref, tbits, ss, n):
    signmask = jnp.int32(0x7FFFFFFF)

    ssv = jnp.full((16,), ss, jnp.float32)
    y = plsc.bitcast(
        (plsc.bitcast(ssv, jnp.int32) >> 1) + jnp.int32(0x1FBD1DF5), jnp.float32
    )
    for _ in range(3):
        y = 0.5 * (y + ssv / y)
    inv = 1.0 / (y + 1e-6)
    inv = inv[0]

    def scale(i, _):
        base = i * 256
        vs = [row_ref[pl.ds(base + u * 16, 16)] for u in range(16)]
        outs = []
        for v in vs:
            b = plsc.bitcast(v, jnp.int32) & signmask
            outs.append(jnp.where(b >= tbits, v, 0.0) * inv)
        for u, o in enumerate(outs):
            row_ref[pl.ds(base + u * 16, 16)] = o
        return 0

    lax.fori_loop(0, n // 256, scale, 0)


def _make_sc_kernel(m, n, k):
    mesh = plsc.VectorSubcoreMesh(core_axis_name="c", subcore_axis_name="s")
    rows_per = m // 32

    @functools.partial(
        pl.kernel,
        mesh=mesh,
        out_type=jax.ShapeDtypeStruct((m, n), jnp.float32),
        compiler_params=pltpu.CompilerParams(needs_layout_passes=False),
        scratch_types=[
            pltpu.VMEM((n,), jnp.float32),
            pltpu.VMEM((n,), jnp.float32),
            pltpu.VMEM((_HISTW,), jnp.int32),
            pltpu.VMEM((n + 128,), jnp.int32),
            pltpu.SemaphoreType.DMA,
            pltpu.SemaphoreType.DMA,
            pltpu.SemaphoreType.DMA,
            pltpu.SemaphoreType.DMA,
        ],
    )
    def sc_kernel(x_hbm, out_hbm, row_a, row_b, hist, cand,
                  si_a, si_b, so_a, so_b):
        wid = lax.axis_index("c") * 16 + lax.axis_index("s")
        base = wid * rows_per
        bufs = (row_a, row_b)
        sin = (si_a, si_b)
        sout = (so_a, so_b)
        zeros = jnp.zeros((16,), jnp.int32)

        def clr(i, _):
            for u in range(8):
                hist[pl.ds(i * 128 + u * 16, 16)] = zeros
            return 0

        lax.fori_loop(0, _HISTW // 128, clr, 0)

        h_in = [None] * rows_per
        h_out = [None] * rows_per
        h_in[0] = pltpu.async_copy(x_hbm.at[base], bufs[0], sin[0])
        if rows_per > 1:
            h_in[1] = pltpu.async_copy(x_hbm.at[base + 1], bufs[1], sin[1])
        for j in range(rows_per):
            h_in[j].wait()
            buf = bufs[j % 2]
            t, ss = _row_thresh_ss(buf, hist, cand, k, n)
            _mask_scale_row(buf, t, ss, n)
            h_out[j] = pltpu.async_copy(buf, out_hbm.at[base + j], sout[j % 2])
            if j + 2 < rows_per:
                h_out[j].wait()
                h_in[j + 2] = pltpu.async_copy(
                    x_hbm.at[base + j + 2], bufs[j % 2], sin[j % 2]
                )
        for j in range(max(0, rows_per - 2), rows_per):
            h_out[j].wait()

    return sc_kernel


@jax.jit
def kernel(x):
    m, n = x.shape
    k = int(_K_RATIO * n)
    return _make_sc_kernel(m, n, k)(x)


# speedup vs baseline: 3.8583x; 1.0829x over previous
import functools

import jax
import jax.numpy as jnp
from jax import lax
from jax.experimental import pallas as pl
from jax.experimental.pallas import tpu as pltpu
from jax.experimental.pallas import tpu_sc as plsc

_K_RATIO = 0.1

_NBINS0 = 1024
_HISTW = 16 * _NBINS0
_SENT = 0x7FFFFFFF
_SMALL_PASSES = ((13, 8, 256), (5, 8, 256), (0, 5, 32))


def _select_pass(read_vreg, ngroups, unroll, hist_ref, kk, prefix, sh, nb,
                 nbins, lane, ones, zeros):
    def scat(i, _):
        bs = [read_vreg(i, u) for u in range(unroll)]
        idxs, ms = [], []
        for b in bs:
            hi = b >> sh
            idxs.append((((hi & jnp.int32((1 << nb) - 1)) << 4)) | lane)
            ms.append((hi >> nb) == prefix)
        for idx, m in zip(idxs, ms):
            plsc.addupdate_scatter(hist_ref, [idx], ones, mask=m)
        return 0

    lax.fori_loop(0, ngroups, scat, 0)

    def scan_b(i, st):
        carry, bcount, gtb = st
        for u in range(4):
            c = jnp.int32(nbins - 1) - (i * 4 + u)
            v = hist_ref[pl.ds(c * 16, 16)]
            hist_ref[pl.ds(c * 16, 16)] = zeros
            carry = carry + jnp.sum(v)
            bcount = bcount + (carry >= kk).astype(jnp.int32)
            gtb = jnp.maximum(gtb, jnp.where(carry < kk, carry, 0))
        return carry, bcount, gtb

    _, bcount, gtb = lax.fori_loop(
        0, nbins // 4, scan_b,
        (jnp.int32(0), jnp.int32(0), jnp.int32(0)),
    )
    return kk - gtb, (prefix << nb) | (bcount - 1)


def _row_thresh_ss(row_ref, hist_ref, cand_ref, k, n, pre_compact):
    lane = lax.iota(jnp.int32, 16)
    ones = jnp.ones((16,), jnp.int32)
    zeros = jnp.zeros((16,), jnp.int32)
    fzeros = jnp.zeros((16,), jnp.float32)
    kk = jnp.int32(k)
    prefix = jnp.int32(0)

    def read_row(i, u):
        v = row_ref[pl.ds(i * 256 + u * 16, 16)]
        return plsc.bitcast(v, jnp.int32) & jnp.int32(0x7FFFFFFF)

    kk, prefix = _select_pass(
        read_row, n // 256, 16, hist_ref, kk, prefix, 21, 10, _NBINS0,
        lane, ones, zeros
    )
    pre_compact()

    def cpt(i, st):
        off = st[0]
        accs = st[1:]
        bs = [
            plsc.bitcast(row_ref[pl.ds(i * 128 + u * 16, 16)], jnp.int32)
            & jnp.int32(0x7FFFFFFF)
            for u in range(8)
        ]
        ms = [(b >> 21) == prefix for b in bs]
        pcs = [plsc.all_reduce_population_count(m)[0] for m in ms]
        offs = []
        for pc in pcs:
            offs.append(off)
            off = off + pc
        new = []
        for b, a in zip(bs, accs):
            hi = plsc.bitcast(b, jnp.float32)
            xm = jnp.where((b >> 21) > prefix, hi, 0.0)
            new.append(a + xm * xm)
        for b, m, o in zip(bs, ms, offs):
            plsc.store_compressed(
                cand_ref.at[pl.ds(o, 16)], plsc.bitcast(b, jnp.float32),
                mask=m,
            )
        return (off, *new)

    st = lax.fori_loop(0, n // 128, cpt, (jnp.int32(0),) + (fzeros,) * 8)
    c1 = st[0]
    ss_hi = st[1] + st[2] + st[3] + st[4] + st[5] + st[6] + st[7] + st[8]
    sent = plsc.bitcast(jnp.full((16,), _SENT, jnp.int32), jnp.float32)
    for u in range(8):
        cand_ref[pl.ds(c1 + u * 16, 16)] = sent
    nit = (c1 + jnp.int32(127)) >> 7

    def read_cand(i, u):
        return plsc.bitcast(cand_ref[pl.ds(i * 128 + u * 16, 16)], jnp.int32)

    for sh, nb, nbins in _SMALL_PASSES:
        kk, prefix = _select_pass(
            read_cand, nit, 8, hist_ref, kk, prefix, sh, nb, nbins, lane,
            ones, zeros
        )

    def cssq(i, accs):
        vs = [cand_ref[pl.ds(i * 128 + u * 16, 16)] for u in range(8)]
        new = []
        for v, a in zip(vs, accs):
            b = plsc.bitcast(v, jnp.int32)
            keep = (b >= prefix) & (b < jnp.int32(_SENT))
            xm = jnp.where(keep, v, 0.0)
            new.append(a + xm * xm)
        return tuple(new)

    accs = lax.fori_loop(0, nit, cssq, (fzeros,) * 8)
    ss_cand = (accs[0] + accs[1] + accs[2] + accs[3]
               + accs[4] + accs[5] + accs[6] + accs[7])
    return prefix, jnp.sum(ss_hi + ss_cand)


def _mask_scale_row(row_ref, out_ref, tbits, ss, n):
    signmask = jnp.int32(0x7FFFFFFF)

    ssv = jnp.full((16,), ss, jnp.float32)
    y = plsc.bitcast(
        (plsc.bitcast(ssv, jnp.int32) >> 1) + jnp.int32(0x1FBD1DF5), jnp.float32
    )
    for _ in range(3):
        y = 0.5 * (y + ssv / y)
    inv = 1.0 / (y + 1e-6)
    inv = inv[0]

    def scale(i, _):
        base = i * 256
        vs = [row_ref[pl.ds(base + u * 16, 16)] for u in range(16)]
        outs = []
        for v in vs:
            b = plsc.bitcast(v, jnp.int32) & signmask
            outs.append(jnp.where(b >= tbits, v, 0.0) * inv)
        for u, o in enumerate(outs):
            out_ref[pl.ds(base + u * 16, 16)] = o
        return 0

    lax.fori_loop(0, n // 256, scale, 0)


def _make_sc_kernel(m, n, k):
    mesh = plsc.VectorSubcoreMesh(core_axis_name="c", subcore_axis_name="s")
    rows_per = m // 32

    @functools.partial(
        pl.kernel,
        mesh=mesh,
        out_type=jax.ShapeDtypeStruct((m, n), jnp.float32),
        compiler_params=pltpu.CompilerParams(needs_layout_passes=False),
        scratch_types=[
            pltpu.VMEM((n,), jnp.float32),
            pltpu.VMEM((n,), jnp.float32),
            pltpu.VMEM((_HISTW,), jnp.int32),
            pltpu.VMEM((n + 128,), jnp.float32),
            pltpu.SemaphoreType.DMA,
            pltpu.SemaphoreType.DMA,
            pltpu.SemaphoreType.DMA,
        ],
    )
    def sc_kernel(x_hbm, out_hbm, row_a, row_b, hist, cand, si_a, si_b, so):
        wid = lax.axis_index("c") * 16 + lax.axis_index("s")
        base = wid * rows_per
        bufs = (row_a, row_b)
        sin = (si_a, si_b)
        zeros = jnp.zeros((16,), jnp.int32)

        def clr(i, _):
            for u in range(8):
                hist[pl.ds(i * 128 + u * 16, 16)] = zeros
            return 0

        lax.fori_loop(0, _HISTW // 128, clr, 0)

        h_in = [None] * rows_per
        h_out = [None] * rows_per
        h_in[0] = pltpu.async_copy(x_hbm.at[base], bufs[0], sin[0])
        if rows_per > 1:
            h_in[1] = pltpu.async_copy(x_hbm.at[base + 1], bufs[1], sin[1])
        for j in range(rows_per):
            h_in[j].wait()
            buf = bufs[j % 2]
            drain = (lambda h: (lambda: h.wait()))(h_out[j - 1]) \
                if j > 0 else (lambda: None)
            t, ss = _row_thresh_ss(buf, hist, cand, k, n, drain)
            _mask_scale_row(buf, cand, t, ss, n)
            if j + 2 < rows_per:
                h_in[j + 2] = pltpu.async_copy(
                    x_hbm.at[base + j + 2], buf, sin[j % 2]
                )
            h_out[j] = pltpu.async_copy(
                cand.at[pl.ds(0, n)], out_hbm.at[base + j], so
            )
        h_out[rows_per - 1].wait()

    return sc_kernel


@jax.jit
def kernel(x):
    m, n = x.shape
    k = int(_K_RATIO * n)
    return _make_sc_kernel(m, n, k)(x)


# speedup vs baseline: 3.9971x; 1.0360x over previous
import functools

import jax
import jax.numpy as jnp
from jax import lax
from jax.experimental import pallas as pl
from jax.experimental.pallas import tpu as pltpu
from jax.experimental.pallas import tpu_sc as plsc

_K_RATIO = 0.1

_NBINS0 = 1024
_HISTW = 16 * _NBINS0
_SENT = 0x7FFFFFFF
_SMALL_PASSES = ((13, 8, 256), (5, 8, 256), (0, 5, 32))


def _select_pass(read_vreg, ngroups, unroll, hist_ref, kk, prefix, sh, nb,
                 nbins, lane, ones, zeros):
    def scat(i, _):
        bs = [read_vreg(i, u) for u in range(unroll)]
        idxs, ms = [], []
        for b in bs:
            hi = b >> sh
            idxs.append((((hi & jnp.int32((1 << nb) - 1)) << 4)) | lane)
            ms.append((hi >> nb) == prefix)
        for idx, m in zip(idxs, ms):
            plsc.addupdate_scatter(hist_ref, [idx], ones, mask=m)
        return 0

    lax.fori_loop(0, ngroups, scat, 0)

    def scan_b(i, st):
        carry, bcount, gtb = st
        for u in range(4):
            c = jnp.int32(nbins - 1) - (i * 4 + u)
            v = hist_ref[pl.ds(c * 16, 16)]
            hist_ref[pl.ds(c * 16, 16)] = zeros
            carry = carry + jnp.sum(v)
            bcount = bcount + (carry >= kk).astype(jnp.int32)
            gtb = jnp.maximum(gtb, jnp.where(carry < kk, carry, 0))
        return carry, bcount, gtb

    _, bcount, gtb = lax.fori_loop(
        0, nbins // 4, scan_b,
        (jnp.int32(0), jnp.int32(0), jnp.int32(0)),
    )
    return kk - gtb, (prefix << nb) | (bcount - 1)


def _row_thresh_ss(row_ref, hist_ref, cand_ref, k, n, pre_compact):
    lane = lax.iota(jnp.int32, 16)
    ones = jnp.ones((16,), jnp.int32)
    zeros = jnp.zeros((16,), jnp.int32)
    fzeros = jnp.zeros((16,), jnp.float32)
    kk = jnp.int32(k)
    prefix = jnp.int32(0)

    def read_row(i, u):
        v = row_ref[pl.ds(i * 256 + u * 16, 16)]
        return plsc.bitcast(v, jnp.int32) & jnp.int32(0x7FFFFFFF)

    kk, prefix = _select_pass(
        read_row, n // 256, 16, hist_ref, kk, prefix, 21, 10, _NBINS0,
        lane, ones, zeros
    )
    pre_compact()

    def cpt(i, st):
        off = st[0]
        accs = st[1:]
        bs = [
            plsc.bitcast(row_ref[pl.ds(i * 256 + u * 16, 16)], jnp.int32)
            & jnp.int32(0x7FFFFFFF)
            for u in range(16)
        ]
        ms = [(b >> 21) == prefix for b in bs]
        pcs = [plsc.all_reduce_population_count(m)[0] for m in ms]
        offs = []
        for pc in pcs:
            offs.append(off)
            off = off + pc
        new = list(accs)
        for u, b in enumerate(bs):
            hi = plsc.bitcast(b, jnp.float32)
            xm = jnp.where((b >> 21) > prefix, hi, 0.0)
            new[u % 8] = new[u % 8] + xm * xm
        for b, m, o in zip(bs, ms, offs):
            plsc.store_compressed(
                cand_ref.at[pl.ds(o, 16)], plsc.bitcast(b, jnp.float32),
                mask=m,
            )
        return (off, *new)

    st = lax.fori_loop(0, n // 256, cpt, (jnp.int32(0),) + (fzeros,) * 8)
    c1 = st[0]
    ss_hi = st[1] + st[2] + st[3] + st[4] + st[5] + st[6] + st[7] + st[8]
    sent = plsc.bitcast(jnp.full((16,), _SENT, jnp.int32), jnp.float32)
    for u in range(8):
        cand_ref[pl.ds(c1 + u * 16, 16)] = sent
    nit = (c1 + jnp.int32(127)) >> 7

    def read_cand(i, u):
        return plsc.bitcast(cand_ref[pl.ds(i * 128 + u * 16, 16)], jnp.int32)

    for sh, nb, nbins in _SMALL_PASSES:
        kk, prefix = _select_pass(
            read_cand, nit, 8, hist_ref, kk, prefix, sh, nb, nbins, lane,
            ones, zeros
        )

    def cssq(i, accs):
        vs = [cand_ref[pl.ds(i * 128 + u * 16, 16)] for u in range(8)]
        new = []
        for v, a in zip(vs, accs):
            b = plsc.bitcast(v, jnp.int32)
            keep = (b >= prefix) & (b < jnp.int32(_SENT))
            xm = jnp.where(keep, v, 0.0)
            new.append(a + xm * xm)
        return tuple(new)

    accs = lax.fori_loop(0, nit, cssq, (fzeros,) * 8)
    ss_cand = (accs[0] + accs[1] + accs[2] + accs[3]
               + accs[4] + accs[5] + accs[6] + accs[7])
    return prefix, jnp.sum(ss_hi + ss_cand)


def _mask_scale_row(row_ref, out_ref, tbits, ss, n):
    signmask = jnp.int32(0x7FFFFFFF)

    ssv = jnp.full((16,), ss, jnp.float32)
    y = plsc.bitcast(
        (plsc.bitcast(ssv, jnp.int32) >> 1) + jnp.int32(0x1FBD1DF5), jnp.float32
    )
    for _ in range(3):
        y = 0.5 * (y + ssv / y)
    inv = 1.0 / (y + 1e-6)
    inv = inv[0]

    def scale(i, _):
        base = i * 256
        vs = [row_ref[pl.ds(base + u * 16, 16)] for u in range(16)]
        outs = []
        for v in vs:
            b = plsc.bitcast(v, jnp.int32) & signmask
            outs.append(jnp.where(b >= tbits, v, 0.0) * inv)
        for u, o in enumerate(outs):
            out_ref[pl.ds(base + u * 16, 16)] = o
        return 0

    lax.fori_loop(0, n // 256, scale, 0)


def _make_sc_kernel(m, n, k):
    mesh = plsc.VectorSubcoreMesh(core_axis_name="c", subcore_axis_name="s")
    rows_per = m // 32

    @functools.partial(
        pl.kernel,
        mesh=mesh,
        out_type=jax.ShapeDtypeStruct((m, n), jnp.float32),
        compiler_params=pltpu.CompilerParams(needs_layout_passes=False),
        scratch_types=[
            pltpu.VMEM((n,), jnp.float32),
            pltpu.VMEM((n,), jnp.float32),
            pltpu.VMEM((_HISTW,), jnp.int32),
            pltpu.VMEM((n + 128,), jnp.float32),
            pltpu.SemaphoreType.DMA,
            pltpu.SemaphoreType.DMA,
            pltpu.SemaphoreType.DMA,
        ],
    )
    def sc_kernel(x_hbm, out_hbm, row_a, row_b, hist, cand, si_a, si_b, so):
        wid = lax.axis_index("c") * 16 + lax.axis_index("s")
        base = wid * rows_per
        bufs = (row_a, row_b)
        sin = (si_a, si_b)
        zeros = jnp.zeros((16,), jnp.int32)

        def clr(i, _):
            for u in range(8):
                hist[pl.ds(i * 128 + u * 16, 16)] = zeros
            return 0

        lax.fori_loop(0, _HISTW // 128, clr, 0)

        h_in = [None] * rows_per
        h_out = [None] * rows_per
        h_in[0] = pltpu.async_copy(x_hbm.at[base], bufs[0], sin[0])
        if rows_per > 1:
            h_in[1] = pltpu.async_copy(x_hbm.at[base + 1], bufs[1], sin[1])
        for j in range(rows_per):
            h_in[j].wait()
            buf = bufs[j % 2]
            drain = (lambda h: (lambda: h.wait()))(h_out[j - 1]) \
                if j > 0 else (lambda: None)
            t, ss = _row_thresh_ss(buf, hist, cand, k, n, drain)
            _mask_scale_row(buf, cand, t, ss, n)
            if j + 2 < rows_per:
                h_in[j + 2] = pltpu.async_copy(
                    x_hbm.at[base + j + 2], buf, sin[j % 2]
                )
            h_out[j] = pltpu.async_copy(
                cand.at[pl.ds(0, n)], out_hbm.at[base + j], so
            )
        h_out[rows_per - 1].wait()

    return sc_kernel


@jax.jit
def kernel(x):
    m, n = x.shape
    k = int(_K_RATIO * n)
    return _make_sc_kernel(m, n, k)(x)
